# baseline (speedup 1.0000x reference)
import jax
import jax.numpy as jnp
from jax import lax
from jax.experimental import pallas as pl
from jax.experimental.pallas import tpu as pltpu
from jax.experimental.pallas import tpu_sc as plsc

B, C, H, W = 8, 19, 512, 512
SC_B = 5
NW = 32
CW = 256
ROWS = 8
HB_PER_BATCH = (H // ROWS) * (W // CW)
NCHUNK = SC_B * HB_PER_BATCH // NW
LANES = 16


def _take16(vec, idx):
    dnums = lax.GatherDimensionNumbers(
        offset_dims=(), collapsed_slice_dims=(0,), start_index_map=(0,))
    return lax.gather(vec, idx[:, None], dnums, slice_sizes=(1,),
                      mode=lax.GatherScatterMode.PROMISE_IN_BOUNDS)


def _sc_body(x_hbm, tab_hbm, out_hbm, in_v, out_v, tab_v, sem0, sem1):
    cid = lax.axis_index("c")
    sid = lax.axis_index("s")
    wid = sid * 2 + cid
    sems = (sem0, sem1)

    def coords(k):
        hb = wid * NCHUNK + k
        b = hb // HB_PER_BATCH
        rem = hb % HB_PER_BATCH
        return b, (rem // 2) * ROWS, (rem % 2) * CW

    def in_copy(k, slot):
        b, r0, w0 = coords(k)
        return pltpu.make_async_copy(
            x_hbm.at[b, :, pl.ds(r0, ROWS), pl.ds(w0, CW)],
            in_v.at[slot], sems[slot])

    pltpu.sync_copy(tab_hbm, tab_v)
    mean_a = tab_v[0, pl.ds(0, LANES)]
    mean_b = tab_v[1, pl.ds(0, LANES)]
    inv_a = tab_v[2, pl.ds(0, LANES)]
    inv_b = tab_v[3, pl.ds(0, LANES)]

    in_copy(0, 0).start()

    def pair_body(g, _):
        for slot in (0, 1):
            k = g * 2 + slot
            in_copy(k, slot).wait()

            @pl.when(k + 1 < NCHUNK)
            def _():
                in_copy(k + 1, 1 - slot).start()

            @plsc.parallel_loop(0, ROWS * CW // LANES, unroll=4)
            def vec_body(i):
                r = i >> 4
                w = (i & 15) * LANES

                def mk_idx(c):
                    return jnp.full((LANES,), c, jnp.int32)

                nodes = [(in_v[slot, c, r, pl.ds(w, LANES)], c)
                         for c in range(C)]
                while len(nodes) > 1:
                    nxt = []
                    for j in range(0, len(nodes) - 1, 2):
                        va, ia = nodes[j]
                        vb, ib = nodes[j + 1]
                        ia = mk_idx(ia) if isinstance(ia, int) else ia
                        ib = mk_idx(ib) if isinstance(ib, int) else ib
                        m = vb > va
                        nxt.append((jnp.maximum(va, vb), jnp.where(m, ib, ia)))
                    if len(nodes) % 2:
                        nxt.append(nodes[-1])
                    nodes = nxt
                best, idx = nodes[0]
                lo = idx < 16
                m15 = idx & 15
                mean = jnp.where(lo, _take16(mean_a, m15), _take16(mean_b, m15))
                inv = jnp.where(lo, _take16(inv_a, m15), _take16(inv_b, m15))
                out_v[r, pl.ds(w, LANES)] = (best - mean) * inv

            b, r0, w0 = coords(k)
            pltpu.sync_copy(
                out_v, out_hbm.at[b, pl.ds(r0, ROWS), pl.ds(w0, CW)])
        return 0

    lax.fori_loop(0, NCHUNK // 2, pair_body, 0)


def _tc_body(means_ref, inv_ref, x_ref, o_ref):
    nodes = [(x_ref[0, c], means_ref[c], inv_ref[c]) for c in range(C)]
    while len(nodes) > 1:
        nxt = []
        for j in range(0, len(nodes) - 1, 2):
            va, ma, ia = nodes[j]
            vb, mb, ib = nodes[j + 1]
            m = vb > va
            nxt.append((jnp.maximum(va, vb), jnp.where(m, mb, ma),
                        jnp.where(m, ib, ia)))
        if len(nodes) % 2:
            nxt.append(nodes[-1])
        nodes = nxt
    best, mean, inv = nodes[0]
    o_ref[0] = (best - mean) * inv


@jax.jit
def kernel(x, means, std):
    inv = 1.0 / std
    tab = jnp.zeros((8, 128), jnp.float32)
    tab = tab.at[0, :16].set(means[:16])
    tab = tab.at[1, :3].set(means[16:])
    tab = tab.at[2, :16].set(inv[:16])
    tab = tab.at[3, :3].set(inv[16:])

    mesh = plsc.VectorSubcoreMesh(core_axis_name="c", subcore_axis_name="s")
    sc_run = pl.kernel(
        _sc_body,
        out_type=jax.ShapeDtypeStruct((SC_B, H, W), jnp.float32),
        mesh=mesh,
        compiler_params=pltpu.CompilerParams(use_tc_tiling_on_sc=True),
        scratch_types=[
            pltpu.VMEM((2, C, ROWS, CW), jnp.float32),
            pltpu.VMEM((ROWS, CW), jnp.float32),
            pltpu.VMEM((8, 128), jnp.float32),
            pltpu.SemaphoreType.DMA,
            pltpu.SemaphoreType.DMA,
        ],
    )
    sc_out = sc_run(x, tab)

    tc_run = pl.pallas_call(
        _tc_body,
        grid=(B - SC_B, H // 64),
        in_specs=[
            pl.BlockSpec(memory_space=pltpu.SMEM),
            pl.BlockSpec(memory_space=pltpu.SMEM),
            pl.BlockSpec((1, C, 64, W), lambda i, j: (i + SC_B, 0, j, 0)),
        ],
        out_specs=pl.BlockSpec((1, 64, W), lambda i, j: (i + SC_B, j, 0)),
        out_shape=jax.ShapeDtypeStruct((B, H, W), jnp.float32),
        compiler_params=pltpu.CompilerParams(
            dimension_semantics=("parallel", "arbitrary")),
    )
    tc_out = tc_run(means, inv, x)
    return lax.dynamic_update_slice(tc_out, sc_out, (0, 0, 0))


# speedup vs baseline: 1.0942x; 1.0942x over previous
import jax
import jax.numpy as jnp
from jax import lax
from jax.experimental import pallas as pl
from jax.experimental.pallas import tpu as pltpu
from jax.experimental.pallas import tpu_sc as plsc

B, C, H, W = 8, 19, 512, 512
SC_B = 4
NW = 32
CW = 256
ROWS = 8
HB_PER_BATCH = (H // ROWS) * (W // CW)
NCHUNK = SC_B * HB_PER_BATCH // NW
LANES = 16


def _take16(vec, idx):
    dnums = lax.GatherDimensionNumbers(
        offset_dims=(), collapsed_slice_dims=(0,), start_index_map=(0,))
    return lax.gather(vec, idx[:, None], dnums, slice_sizes=(1,),
                      mode=lax.GatherScatterMode.PROMISE_IN_BOUNDS)


def _sc_body(x_hbm, tab_hbm, out_hbm, in_v, out_v, tab_v, sem0, sem1):
    cid = lax.axis_index("c")
    sid = lax.axis_index("s")
    wid = sid * 2 + cid
    sems = (sem0, sem1)

    def coords(k):
        hb = wid * NCHUNK + k
        b = hb // HB_PER_BATCH
        rem = hb % HB_PER_BATCH
        return b, (rem // 2) * ROWS, (rem % 2) * CW

    def in_copy(k, slot):
        b, r0, w0 = coords(k)
        return pltpu.make_async_copy(
            x_hbm.at[b, :, pl.ds(r0, ROWS), pl.ds(w0, CW)],
            in_v.at[slot], sems[slot])

    pltpu.sync_copy(tab_hbm, tab_v)
    mean_a = tab_v[0, pl.ds(0, LANES)]
    mean_b = tab_v[1, pl.ds(0, LANES)]
    inv_a = tab_v[2, pl.ds(0, LANES)]
    inv_b = tab_v[3, pl.ds(0, LANES)]

    in_copy(0, 0).start()

    def pair_body(g, _):
        for slot in (0, 1):
            k = g * 2 + slot
            in_copy(k, slot).wait()

            @pl.when(k + 1 < NCHUNK)
            def _():
                in_copy(k + 1, 1 - slot).start()

            @plsc.parallel_loop(0, ROWS * CW // LANES, unroll=4)
            def vec_body(i):
                r = i >> 4
                w = (i & 15) * LANES

                def mk_idx(c):
                    return jnp.full((LANES,), c, jnp.int32)

                nodes = [(in_v[slot, c, r, pl.ds(w, LANES)], c)
                         for c in range(C)]
                while len(nodes) > 1:
                    nxt = []
                    for j in range(0, len(nodes) - 1, 2):
                        va, ia = nodes[j]
                        vb, ib = nodes[j + 1]
                        ia = mk_idx(ia) if isinstance(ia, int) else ia
                        ib = mk_idx(ib) if isinstance(ib, int) else ib
                        m = vb > va
                        nxt.append((jnp.maximum(va, vb), jnp.where(m, ib, ia)))
                    if len(nodes) % 2:
                        nxt.append(nodes[-1])
                    nodes = nxt
                best, idx = nodes[0]
                lo = idx < 16
                m15 = idx & 15
                mean = jnp.where(lo, _take16(mean_a, m15), _take16(mean_b, m15))
                inv = jnp.where(lo, _take16(inv_a, m15), _take16(inv_b, m15))
                out_v[r, pl.ds(w, LANES)] = (best - mean) * inv

            b, r0, w0 = coords(k)
            pltpu.sync_copy(
                out_v, out_hbm.at[b, pl.ds(r0, ROWS), pl.ds(w0, CW)])
        return 0

    lax.fori_loop(0, NCHUNK // 2, pair_body, 0)


def _tc_body(means_ref, inv_ref, x_ref, o_ref):
    nodes = [(x_ref[0, c], means_ref[c], inv_ref[c]) for c in range(C)]
    while len(nodes) > 1:
        nxt = []
        for j in range(0, len(nodes) - 1, 2):
            va, ma, ia = nodes[j]
            vb, mb, ib = nodes[j + 1]
            m = vb > va
            nxt.append((jnp.maximum(va, vb), jnp.where(m, mb, ma),
                        jnp.where(m, ib, ia)))
        if len(nodes) % 2:
            nxt.append(nodes[-1])
        nodes = nxt
    best, mean, inv = nodes[0]
    o_ref[0] = (best - mean) * inv


@jax.jit
def kernel(x, means, std):
    inv = 1.0 / std
    tab = jnp.zeros((8, 128), jnp.float32)
    tab = tab.at[0, :16].set(means[:16])
    tab = tab.at[1, :3].set(means[16:])
    tab = tab.at[2, :16].set(inv[:16])
    tab = tab.at[3, :3].set(inv[16:])

    mesh = plsc.VectorSubcoreMesh(core_axis_name="c", subcore_axis_name="s")
    sc_run = pl.kernel(
        _sc_body,
        out_type=jax.ShapeDtypeStruct((SC_B, H, W), jnp.float32),
        mesh=mesh,
        compiler_params=pltpu.CompilerParams(use_tc_tiling_on_sc=True),
        scratch_types=[
            pltpu.VMEM((2, C, ROWS, CW), jnp.float32),
            pltpu.VMEM((ROWS, CW), jnp.float32),
            pltpu.VMEM((8, 128), jnp.float32),
            pltpu.SemaphoreType.DMA,
            pltpu.SemaphoreType.DMA,
        ],
    )
    sc_out = sc_run(x, tab)

    tc_run = pl.pallas_call(
        _tc_body,
        grid=(B - SC_B, H // 64),
        in_specs=[
            pl.BlockSpec(memory_space=pltpu.SMEM),
            pl.BlockSpec(memory_space=pltpu.SMEM),
            pl.BlockSpec((1, C, 64, W), lambda i, j: (i + SC_B, 0, j, 0)),
        ],
        out_specs=pl.BlockSpec((1, 64, W), lambda i, j: (i + SC_B, j, 0)),
        out_shape=jax.ShapeDtypeStruct((B, H, W), jnp.float32),
        compiler_params=pltpu.CompilerParams(
            dimension_semantics=("parallel", "arbitrary")),
    )
    tc_out = tc_run(means, inv, x)
    return lax.dynamic_update_slice(tc_out, sc_out, (0, 0, 0))
